# baseline (speedup 1.0000x reference)
import jax
import jax.numpy as jnp
from jax.experimental import pallas as pl
from jax.experimental.pallas import tpu as pltpu

_OLD_W, _OLD_H = 224, 224
_NEW_W, _NEW_H = 28, 28
_CELLS = _NEW_W * _NEW_H


def _scatter_kernel(p_ref, enc_ref, out_ref, scratch):
    b = pl.program_id(0)
    scratch[...] = jnp.zeros(scratch.shape, scratch.dtype)
    e = enc_ref[0]
    n_sv = e.shape[0]
    for n in range(n_sv):
        pn = p_ref[b, n]
        scratch[pn, 0] = jnp.maximum(scratch[pn, 0], e[n])
    t = scratch[:_CELLS, 0, :].T
    out_ref[0] = t.reshape(t.shape[0], _NEW_W, _NEW_H)


def kernel(svPositionsAtT0, svEncoding, lengths):
    b_, n_, hid = svEncoding.shape
    x = svPositionsAtT0[..., 0]
    y = svPositionsAtT0[..., 1]
    xIdx = (x * _NEW_W / _OLD_W).astype(jnp.int32)
    yIdx = (y * _NEW_H / _OLD_H).astype(jnp.int32)
    agent_ids = jnp.arange(n_, dtype=lengths.dtype)[None, :]
    valid = (lengths[:, None] > agent_ids) & (xIdx < _NEW_W) & (yIdx < _NEW_H)
    xI = jnp.clip(xIdx, 0, _NEW_W - 1)
    yI = jnp.clip(yIdx, 0, _NEW_H - 1)
    p = jnp.where(valid, xI * _NEW_H + yI, _CELLS).astype(jnp.int32)

    out = pl.pallas_call(
        _scatter_kernel,
        grid_spec=pltpu.PrefetchScalarGridSpec(
            num_scalar_prefetch=1,
            grid=(b_,),
            in_specs=[pl.BlockSpec((1, n_, hid), lambda b, pr: (b, 0, 0))],
            out_specs=pl.BlockSpec(
                (1, hid, _NEW_W, _NEW_H), lambda b, pr: (b, 0, 0, 0)
            ),
            scratch_shapes=[pltpu.VMEM((_CELLS + 1, 1, hid), jnp.float32)],
        ),
        out_shape=jax.ShapeDtypeStruct((b_, hid, _NEW_W, _NEW_H), jnp.float32),
        compiler_params=pltpu.CompilerParams(dimension_semantics=("parallel",)),
    )(p, svEncoding)
    return out


# speedup vs baseline: 13.3173x; 13.3173x over previous
import jax
import jax.numpy as jnp
from jax.experimental import pallas as pl
from jax.experimental.pallas import tpu as pltpu

_OLD_W, _OLD_H = 224, 224
_NEW_W, _NEW_H = 28, 28
_CELLS = _NEW_W * _NEW_H
_BG = 8


def _scatter_kernel(p_ref, v_ref, enc_ref, out_ref):
    g = pl.program_id(0)
    out_ref[...] = jnp.zeros(out_ref.shape, out_ref.dtype)
    n_sv = enc_ref.shape[0]
    for n in range(n_sv):
        updates = []
        for b8 in range(_BG):
            pn = p_ref[g * _BG + b8, n]
            vn = v_ref[g * _BG + b8, n].astype(jnp.float32)
            row = enc_ref[n, b8 : b8 + 1, :] * vn
            updates.append((pn, jnp.maximum(out_ref[pn, b8 : b8 + 1, :], row)))
        for b8 in range(_BG):
            pn, val = updates[b8]
            out_ref[pn, b8 : b8 + 1, :] = val


def kernel(svPositionsAtT0, svEncoding, lengths):
    b_, n_, hid = svEncoding.shape
    x = svPositionsAtT0[..., 0]
    y = svPositionsAtT0[..., 1]
    xIdx = (x * _NEW_W / _OLD_W).astype(jnp.int32)
    yIdx = (y * _NEW_H / _OLD_H).astype(jnp.int32)
    agent_ids = jnp.arange(n_, dtype=lengths.dtype)[None, :]
    valid = (lengths[:, None] > agent_ids) & (xIdx < _NEW_W) & (yIdx < _NEW_H)
    xI = jnp.clip(xIdx, 0, _NEW_W - 1)
    yI = jnp.clip(yIdx, 0, _NEW_H - 1)
    p = (xI * _NEW_H + yI).astype(jnp.int32)
    v = valid.astype(jnp.int32)

    enc_t = svEncoding.transpose(1, 0, 2)

    zz = pl.pallas_call(
        _scatter_kernel,
        grid_spec=pltpu.PrefetchScalarGridSpec(
            num_scalar_prefetch=2,
            grid=(b_ // _BG,),
            in_specs=[
                pl.BlockSpec((n_, _BG, hid), lambda g, pr, vr: (0, g, 0)),
            ],
            out_specs=pl.BlockSpec((_CELLS, _BG, hid), lambda g, pr, vr: (0, g, 0)),
        ),
        out_shape=jax.ShapeDtypeStruct((_CELLS, b_, hid), jnp.float32),
        compiler_params=pltpu.CompilerParams(dimension_semantics=("arbitrary",)),
    )(p, v, enc_t)
    return zz.reshape(_NEW_W, _NEW_H, b_, hid).transpose(2, 3, 0, 1)
